# baseline (speedup 1.0000x reference)
import functools

import jax
import jax.numpy as jnp
from jax import lax
from jax.experimental import pallas as pl
from jax.experimental.pallas import tpu as pltpu
from jax.experimental.pallas import tpu_sc as plsc

N_NODES = 10000
N_EDGES = 320000
D_FEAT = 128
DH = D_FEAT // 2
NS = 16
G = 128
NB = 160
NSEG = 4
SEG = NB // NSEG
E_PAD = NS * NB * G
ROWS_PER_SUB = N_NODES // NS


def _gnn_body(xcat, src2, dst2, w2, out, idx_v, dst_v, w_v,
              rows_a, rows_b, msg_v, x_s, acc, sem_a, sem_b):
    cid = lax.axis_index("c")
    sid = lax.axis_index("s")

    pltpu.sync_copy(xcat.at[pl.ds(cid * N_NODES + sid * ROWS_PER_SUB,
                                  ROWS_PER_SUB)],
                    x_s.at[pl.ds(sid * ROWS_PER_SUB, ROWS_PER_SUB)])

    zeros16 = jnp.zeros((16,), jnp.float32)

    def zero_row(i, carry):
        for q in range(DH // 16):
            rows_a[i, pl.ds(q * 16, 16)] = zeros16
        return carry

    lax.fori_loop(0, G, zero_row, None)
    row0 = sid * ROWS_PER_SUB
    for k in range(ROWS_PER_SUB // G):
        pltpu.sync_copy(rows_a, acc.at[pl.ds(row0 + k * G, G)])
    rem = ROWS_PER_SUB % G
    if rem:
        pltpu.sync_copy(rows_a.at[pl.ds(0, rem)],
                        acc.at[pl.ds(row0 + ROWS_PER_SUB - rem, rem)])
    plsc.subcore_barrier()

    def _scale(buf, g):
        for t in range(G // 16):
            wv = w_v[g, pl.ds(t * 16, 16)]
            for j in range(16):
                e = t * 16 + j
                ws = lax.squeeze(lax.slice(wv, (j,), (j + 1,)), (0,))
                wb = lax.broadcast_in_dim(ws, (16,), ())
                for q in range(DH // 16):
                    v = buf[e, pl.ds(q * 16, 16)]
                    msg_v[e, pl.ds(q * 16, 16)] = v * wb

    def start_g(g, buf, sem):
        pltpu.async_copy(x_s.at[idx_v.at[g]], buf, sem)

    def wait_g(g, buf, sem):
        pltpu.make_async_copy(x_s.at[idx_v.at[g]], buf, sem).wait()

    for seg in range(NSEG):
        pltpu.sync_copy(src2.at[sid, pl.ds(seg * SEG, SEG)], idx_v)
        pltpu.sync_copy(dst2.at[sid, pl.ds(seg * SEG, SEG)], dst_v)
        pltpu.sync_copy(w2.at[sid, pl.ds(seg * SEG, SEG)], w_v)

        start_g(0, rows_a, sem_a)

        def blk2(i, carry):
            ga = 2 * i
            gb = 2 * i + 1
            wait_g(ga, rows_a, sem_a)
            start_g(gb, rows_b, sem_b)
            _scale(rows_a, ga)
            pltpu.sync_copy(msg_v, acc.at[dst_v.at[ga]], add=True)

            wait_g(gb, rows_b, sem_b)
            gn = jnp.minimum(gb + 1, SEG - 1)
            start_g(gn, rows_a, sem_a)
            _scale(rows_b, gb)
            pltpu.sync_copy(msg_v, acc.at[dst_v.at[gb]], add=True)
            return carry

        lax.fori_loop(0, SEG // 2, blk2, None)
        wait_g(SEG - 1, rows_a, sem_a)

    plsc.subcore_barrier()

    pltpu.sync_copy(acc.at[pl.ds(row0, ROWS_PER_SUB)],
                    out.at[pl.ds(row0, ROWS_PER_SUB), pl.ds(cid * DH, DH)])


@functools.partial(
    pl.kernel,
    out_type=jax.ShapeDtypeStruct((N_NODES, D_FEAT), jnp.float32),
    mesh=plsc.VectorSubcoreMesh(core_axis_name="c", subcore_axis_name="s"),
    compiler_params=pltpu.CompilerParams(use_tc_tiling_on_sc=False),
    scratch_types=[
        pltpu.VMEM((SEG, G), jnp.int32),
        pltpu.VMEM((SEG, G), jnp.int32),
        pltpu.VMEM((SEG, G), jnp.float32),
        pltpu.VMEM((G, DH), jnp.float32),
        pltpu.VMEM((G, DH), jnp.float32),
        pltpu.VMEM((G, DH), jnp.float32),
        pltpu.VMEM_SHARED((N_NODES, DH), jnp.float32),
        pltpu.VMEM_SHARED((N_NODES, DH), jnp.float32),
        pltpu.SemaphoreType.DMA,
        pltpu.SemaphoreType.DMA,
    ],
)
def _gnn_sc(xcat, src2, dst2, w2, out, idx_v, dst_v, w_v,
            rows_a, rows_b, msg_v, x_s, acc, sem_a, sem_b):
    _gnn_body(xcat, src2, dst2, w2, out, idx_v, dst_v, w_v,
              rows_a, rows_b, msg_v, x_s, acc, sem_a, sem_b)


def kernel(x, edge_index, edge_weight):
    xcat = jnp.concatenate([x[:, :DH], x[:, DH:]], axis=0)
    pad = E_PAD - N_EDGES
    src = jnp.concatenate(
        [edge_index[0].astype(jnp.int32), jnp.zeros((pad,), jnp.int32)])
    dst = jnp.concatenate(
        [edge_index[1].astype(jnp.int32), jnp.zeros((pad,), jnp.int32)])
    w = jnp.concatenate([edge_weight, jnp.zeros((pad,), jnp.float32)])
    return _gnn_sc(xcat, src.reshape(NS, NB, G), dst.reshape(NS, NB, G),
                   w.reshape(NS, NB, G))


# speedup vs baseline: 1.0715x; 1.0715x over previous
import functools

import jax
import jax.numpy as jnp
from jax import lax
from jax.experimental import pallas as pl
from jax.experimental.pallas import tpu as pltpu
from jax.experimental.pallas import tpu_sc as plsc

N_NODES = 10000
N_EDGES = 320000
D_FEAT = 128
DH = D_FEAT // 2
NS = 16
G = 128
NB = 160
NSEG = 2
SEG = NB // NSEG
E_PAD = NS * NB * G
ROWS_PER_SUB = N_NODES // NS


def _gnn_body(xcat, src2, dst2, w2, out, idx_v, dst_v, w_v,
              rows_a, rows_b, x_s, acc, sem_a, sem_b):
    cid = lax.axis_index("c")
    sid = lax.axis_index("s")

    pltpu.sync_copy(xcat.at[pl.ds(cid * N_NODES + sid * ROWS_PER_SUB,
                                  ROWS_PER_SUB)],
                    x_s.at[pl.ds(sid * ROWS_PER_SUB, ROWS_PER_SUB)])

    zeros16 = jnp.zeros((16,), jnp.float32)

    def zero_row(i, carry):
        for q in range(DH // 16):
            rows_a[i, pl.ds(q * 16, 16)] = zeros16
        return carry

    lax.fori_loop(0, G, zero_row, None)
    row0 = sid * ROWS_PER_SUB
    for k in range(ROWS_PER_SUB // G):
        pltpu.sync_copy(rows_a, acc.at[pl.ds(row0 + k * G, G)])
    rem = ROWS_PER_SUB % G
    if rem:
        pltpu.sync_copy(rows_a.at[pl.ds(0, rem)],
                        acc.at[pl.ds(row0 + ROWS_PER_SUB - rem, rem)])
    plsc.subcore_barrier()

    def _scale(buf, g):
        for t in range(G // 16):
            wv = w_v[g, pl.ds(t * 16, 16)]
            for j in range(16):
                e = t * 16 + j
                ws = lax.squeeze(lax.slice(wv, (j,), (j + 1,)), (0,))
                wb = lax.broadcast_in_dim(ws, (16,), ())
                for q in range(DH // 16):
                    v = buf[e, pl.ds(q * 16, 16)]
                    buf[e, pl.ds(q * 16, 16)] = v * wb

    def start_g(g, buf, sem):
        pltpu.async_copy(x_s.at[idx_v.at[g]], buf, sem)

    def wait_g(g, buf, sem):
        pltpu.make_async_copy(x_s.at[idx_v.at[g]], buf, sem).wait()

    for seg in range(NSEG):
        pltpu.sync_copy(src2.at[sid, pl.ds(seg * SEG, SEG)], idx_v)
        pltpu.sync_copy(dst2.at[sid, pl.ds(seg * SEG, SEG)], dst_v)
        pltpu.sync_copy(w2.at[sid, pl.ds(seg * SEG, SEG)], w_v)

        start_g(0, rows_a, sem_a)

        def blk2(i, carry):
            ga = 2 * i
            gb = 2 * i + 1
            wait_g(ga, rows_a, sem_a)
            start_g(gb, rows_b, sem_b)
            _scale(rows_a, ga)
            pltpu.sync_copy(rows_a, acc.at[dst_v.at[ga]], add=True)

            wait_g(gb, rows_b, sem_b)
            gn = jnp.minimum(gb + 1, SEG - 1)
            start_g(gn, rows_a, sem_a)
            _scale(rows_b, gb)
            pltpu.sync_copy(rows_b, acc.at[dst_v.at[gb]], add=True)
            return carry

        lax.fori_loop(0, SEG // 2, blk2, None)
        wait_g(SEG - 1, rows_a, sem_a)

    plsc.subcore_barrier()

    pltpu.sync_copy(acc.at[pl.ds(row0, ROWS_PER_SUB)],
                    out.at[pl.ds(row0, ROWS_PER_SUB), pl.ds(cid * DH, DH)])


@functools.partial(
    pl.kernel,
    out_type=jax.ShapeDtypeStruct((N_NODES, D_FEAT), jnp.float32),
    mesh=plsc.VectorSubcoreMesh(core_axis_name="c", subcore_axis_name="s"),
    compiler_params=pltpu.CompilerParams(use_tc_tiling_on_sc=False),
    scratch_types=[
        pltpu.VMEM((SEG, G), jnp.int32),
        pltpu.VMEM((SEG, G), jnp.int32),
        pltpu.VMEM((SEG, G), jnp.float32),
        pltpu.VMEM((G, DH), jnp.float32),
        pltpu.VMEM((G, DH), jnp.float32),
        pltpu.VMEM_SHARED((N_NODES, DH), jnp.float32),
        pltpu.VMEM_SHARED((N_NODES, DH), jnp.float32),
        pltpu.SemaphoreType.DMA,
        pltpu.SemaphoreType.DMA,
    ],
)
def _gnn_sc(xcat, src2, dst2, w2, out, idx_v, dst_v, w_v,
            rows_a, rows_b, x_s, acc, sem_a, sem_b):
    _gnn_body(xcat, src2, dst2, w2, out, idx_v, dst_v, w_v,
              rows_a, rows_b, x_s, acc, sem_a, sem_b)


def kernel(x, edge_index, edge_weight):
    xcat = jnp.concatenate([x[:, :DH], x[:, DH:]], axis=0)
    pad = E_PAD - N_EDGES
    src = jnp.concatenate(
        [edge_index[0].astype(jnp.int32), jnp.zeros((pad,), jnp.int32)])
    dst = jnp.concatenate(
        [edge_index[1].astype(jnp.int32), jnp.zeros((pad,), jnp.int32)])
    w = jnp.concatenate([edge_weight, jnp.zeros((pad,), jnp.float32)])
    return _gnn_sc(xcat, src.reshape(NS, NB, G), dst.reshape(NS, NB, G),
                   w.reshape(NS, NB, G))
